# baseline (speedup 1.0000x reference)
import functools

import jax
import jax.numpy as jnp
from jax import lax
from jax.experimental import pallas as pl
from jax.experimental.pallas import tpu as pltpu
from jax.experimental.pallas import tpu_sc as plsc

MAX_LEN = 2048
EMBED_DIM = 768
NUM_CORES = 2
ROWS_PER_CORE = MAX_LEN // NUM_CORES
NCHUNK = 4
CHUNK = ROWS_PER_CORE // NCHUNK

_mesh = plsc.ScalarSubcoreMesh(axis_name="c", num_cores=NUM_CORES)


@functools.partial(
    pl.kernel,
    out_type=jax.ShapeDtypeStruct((MAX_LEN, EMBED_DIM), jnp.float32),
    mesh=_mesh,
    scratch_types=(
        [pltpu.VMEM_SHARED((ROWS_PER_CORE, EMBED_DIM), jnp.float32)]
        + [pltpu.SemaphoreType.DMA] * (2 * NCHUNK)
    ),
)
def _positional_lookup(table_hbm, out_hbm, stage, *sems):
    base = lax.axis_index("c") * ROWS_PER_CORE
    gathers = []
    for i in range(NCHUNK):
        gathers.append(
            pltpu.async_copy(
                table_hbm.at[pl.ds(base + i * CHUNK, CHUNK)],
                stage.at[pl.ds(i * CHUNK, CHUNK)],
                sems[i],
            )
        )
    scatters = []
    for i in range(NCHUNK):
        gathers[i].wait()
        scatters.append(
            pltpu.async_copy(
                stage.at[pl.ds(i * CHUNK, CHUNK)],
                out_hbm.at[pl.ds(base + i * CHUNK, CHUNK)],
                sems[NCHUNK + i],
            )
        )
    for s in scatters:
        s.wait()


def kernel(x, table):
    del x
    return _positional_lookup(table)[None]


# speedup vs baseline: 4.0198x; 4.0198x over previous
import functools

import jax
import jax.numpy as jnp
from jax.experimental import pallas as pl
from jax.experimental.pallas import tpu as pltpu

MAX_LEN = 2048
EMBED_DIM = 768
BLOCK_ROWS = 512


def _copy_body(table_ref, out_ref):
    out_ref[...] = table_ref[...]


@jax.jit
def _tc_copy(table):
    return pl.pallas_call(
        _copy_body,
        grid=(MAX_LEN // BLOCK_ROWS,),
        in_specs=[pl.BlockSpec((BLOCK_ROWS, EMBED_DIM), lambda i: (i, 0))],
        out_specs=pl.BlockSpec((BLOCK_ROWS, EMBED_DIM), lambda i: (i, 0)),
        out_shape=jax.ShapeDtypeStruct((MAX_LEN, EMBED_DIM), jnp.float32),
    )(table)


def kernel(x, table):
    del x
    return _tc_copy(table)[None]


# speedup vs baseline: 5.0180x; 1.2483x over previous
import functools

import jax
import jax.numpy as jnp
from jax.experimental import pallas as pl
from jax.experimental.pallas import tpu as pltpu

MAX_LEN = 2048
EMBED_DIM = 768
BLOCK_ROWS = 1024


def _copy_body(table_ref, out_ref):
    out_ref[...] = table_ref[...]


@jax.jit
def _tc_copy(table):
    return pl.pallas_call(
        _copy_body,
        grid=(MAX_LEN // BLOCK_ROWS,),
        in_specs=[pl.BlockSpec((BLOCK_ROWS, EMBED_DIM), lambda i: (i, 0))],
        out_specs=pl.BlockSpec((BLOCK_ROWS, EMBED_DIM), lambda i: (i, 0)),
        out_shape=jax.ShapeDtypeStruct((MAX_LEN, EMBED_DIM), jnp.float32),
    )(table)


def kernel(x, table):
    del x
    return _tc_copy(table)[None]
